# Initial kernel scaffold; baseline (speedup 1.0000x reference)
import jax
import jax.numpy as jnp
from jax.experimental import pallas as pl


def kernel(x, edge_index, W1, att_src1, att_dst1, bias1, gamma, beta, W2, att_src2, att_dst2, bias2):
    raise NotImplementedError("write your pallas kernel here")


# speedup vs baseline: 29.4908x; 29.4908x over previous
import functools

import jax
import jax.numpy as jnp
from jax import lax
from jax.experimental import pallas as pl
from jax.experimental.pallas import tpu as pltpu
from jax.experimental.pallas import tpu_sc as plsc

N = 10000
E = 320000
EP = 330000
IN_DIM = 128
HID = 64
HEADS = 4
OUT = 2

NPAD = 10240
EPAD = 331776
EPT = EPAD // 16
SUP = 3456
NSUP = EPT // SUP
G = 128
NG = SUP // G
RB = 1280
NRB = NPAD // RB

f32 = jnp.float32
i32 = jnp.int32


def _k1_body(x_ref, w1_ref, asrc_ref, adst_ref, h4_ref, asad_ref):
    h = jnp.dot(x_ref[...], w1_ref[...], preferred_element_type=f32)
    for hh in range(4):
        h4_ref[hh] = h[:, hh * 64:(hh + 1) * 64]
    hr = h.reshape(RB, HEADS, HID)
    a_s = (hr * asrc_ref[...][None]).sum(-1)
    a_d = (hr * adst_ref[...][None]).sum(-1)
    asad_ref[...] = jnp.concatenate([a_s, a_d], axis=1)


def _k1(x_pad, W1, att_src1, att_dst1):
    return pl.pallas_call(
        _k1_body,
        grid=(NRB,),
        in_specs=[
            pl.BlockSpec((RB, IN_DIM), lambda i: (i, 0)),
            pl.BlockSpec((IN_DIM, HEADS * HID), lambda i: (0, 0)),
            pl.BlockSpec((HEADS, HID), lambda i: (0, 0)),
            pl.BlockSpec((HEADS, HID), lambda i: (0, 0)),
        ],
        out_specs=[
            pl.BlockSpec((4, RB, 64), lambda i: (0, i, 0)),
            pl.BlockSpec((RB, 8), lambda i: (i, 0)),
        ],
        out_shape=[
            jax.ShapeDtypeStruct((4, NPAD, 64), f32),
            jax.ShapeDtypeStruct((NPAD, 8), f32),
        ],
    )(x_pad, W1, att_src1, att_dst1)


def _k2_body(j, srcp, dstp, asall, adall, h4, z2d, z1d, outj,
             asb, adb, dnb, sbuf, dbuf, sg, dg,
             exg, ixg, attg, hrows, acc, dnm):
    cid = lax.axis_index("c")
    sid = lax.axis_index("s")
    hsel = 2 * j + cid
    rows_pt = NPAD // 16
    r0 = sid * rows_pt

    pltpu.sync_copy(asall.at[pl.ds(hsel * NPAD, NPAD)], asb)
    pltpu.sync_copy(adall.at[pl.ds(hsel * NPAD, NPAD)], adb)
    pltpu.sync_copy(z2d.at[pl.ds(r0, rows_pt)], acc.at[pl.ds(r0, rows_pt)])
    pltpu.sync_copy(z1d.at[pl.ds(sid * 640, 640)],
                    dnm.at[pl.ds(sid * 640, 640)])
    plsc.subcore_barrier()

    base_t = sid * EPT

    def supA(u, _):
        sb0 = base_t + u * SUP
        pltpu.sync_copy(srcp.at[pl.ds(sb0, SUP)], sbuf)
        pltpu.sync_copy(dstp.at[pl.ds(sb0, SUP)], dbuf)

        def grpA(g, _):
            def vA(v, _):
                off = g * G + v * 16
                si = sbuf[pl.ds(off, 16)]
                di = dbuf[pl.ds(off, 16)]
                a = plsc.load_gather(asb, [si])
                b = plsc.load_gather(adb, [di])
                al = a + b
                al = jnp.maximum(al, 0.2 * al)
                exg[pl.ds(v * 16, 16)] = jnp.exp(al)
                ixg[pl.ds(v * 16, 16)] = di
                return 0

            lax.fori_loop(0, 8, vA, 0)
            pltpu.sync_copy(exg, dnm.at[ixg], add=True)
            return 0

        lax.fori_loop(0, NG, grpA, 0)
        return 0

    lax.fori_loop(0, NSUP, supA, 0)
    plsc.subcore_barrier()
    pltpu.sync_copy(dnm, dnb)

    def supB(u, _):
        sb0 = base_t + u * SUP
        pltpu.sync_copy(srcp.at[pl.ds(sb0, SUP)], sbuf)
        pltpu.sync_copy(dstp.at[pl.ds(sb0, SUP)], dbuf)

        def grpB(g, _):
            def vB(v, _):
                off = g * G + v * 16
                si = sbuf[pl.ds(off, 16)]
                di = dbuf[pl.ds(off, 16)]
                sg[pl.ds(v * 16, 16)] = si + hsel * NPAD
                dg[pl.ds(v * 16, 16)] = di
                a = plsc.load_gather(asb, [si])
                b = plsc.load_gather(adb, [di])
                al = a + b
                al = jnp.maximum(al, 0.2 * al)
                ex = jnp.exp(al)
                dn = plsc.load_gather(dnb, [di])
                attg[pl.ds(v * 16, 16)] = ex / (dn + 1e-16)
                return 0

            lax.fori_loop(0, 8, vB, 0)
            pltpu.sync_copy(h4.at[sg], hrows)

            def eB(e, _):
                zi = jnp.zeros((16,), i32)
                a0 = plsc.load_gather(attg, [zi + e])
                for q in range(4):
                    hv = hrows[e, pl.ds(q * 16, 16)]
                    hrows[e, pl.ds(q * 16, 16)] = hv * a0
                return 0

            lax.fori_loop(0, G, eB, 0)
            pltpu.sync_copy(hrows, acc.at[dg], add=True)
            return 0

        lax.fori_loop(0, NG, grpB, 0)
        return 0

    lax.fori_loop(0, NSUP, supB, 0)
    plsc.subcore_barrier()
    pltpu.sync_copy(acc.at[pl.ds(r0, rows_pt)],
                    outj.at[pl.ds(cid * NPAD + r0, rows_pt)])


def _k2(j, srcp, dstp, as_all, ad_all, h4, z2d, z1d):
    mesh = plsc.VectorSubcoreMesh(core_axis_name="c", subcore_axis_name="s", num_cores=2, num_subcores=16)
    return pl.kernel(
        functools.partial(_k2_body, j),
        out_type=jax.ShapeDtypeStruct((2 * NPAD, 64), f32),
        mesh=mesh,
        compiler_params=pltpu.CompilerParams(needs_layout_passes=False, use_tc_tiling_on_sc=False),
        scratch_types=[
            pltpu.VMEM((NPAD,), f32),
            pltpu.VMEM((NPAD,), f32),
            pltpu.VMEM((NPAD,), f32),
            pltpu.VMEM((SUP,), i32),
            pltpu.VMEM((SUP,), i32),
            pltpu.VMEM((G,), i32),
            pltpu.VMEM((G,), i32),
            pltpu.VMEM((G,), f32),
            pltpu.VMEM((G,), i32),
            pltpu.VMEM((G,), f32),
            pltpu.VMEM((G, 64), f32),
            pltpu.VMEM_SHARED((NPAD, 64), f32),
            pltpu.VMEM_SHARED((NPAD,), f32),
        ],
    )(srcp, dstp, as_all, ad_all, h4, z2d, z1d)


def _k3a_body(oa0_ref, oa1_ref, ob0_ref, ob1_ref, b1_ref, stats_ref):
    i = pl.program_id(0)
    x1 = jnp.concatenate([oa0_ref[...], oa1_ref[...], ob0_ref[...],
                          ob1_ref[...]], axis=1) + b1_ref[...]
    row = i * RB + lax.broadcasted_iota(i32, (RB, 1), 0)
    x1 = jnp.where(row < N, x1, 0.0)
    s1 = jnp.sum(x1, axis=0, keepdims=True)
    s2 = jnp.sum(x1 * x1, axis=0, keepdims=True)
    blk = jnp.concatenate([s1, s2], axis=0)

    @pl.when(i == 0)
    def _():
        stats_ref[...] = jnp.zeros_like(stats_ref)

    stats_ref[...] += blk


def _k3a(oa, ob, bias1_2d):
    return pl.pallas_call(
        _k3a_body,
        grid=(NRB,),
        in_specs=[
            pl.BlockSpec((RB, 64), lambda i: (i, 0)),
            pl.BlockSpec((RB, 64), lambda i: (NRB + i, 0)),
            pl.BlockSpec((RB, 64), lambda i: (i, 0)),
            pl.BlockSpec((RB, 64), lambda i: (NRB + i, 0)),
            pl.BlockSpec((1, 256), lambda i: (0, 0)),
        ],
        out_specs=pl.BlockSpec((2, 256), lambda i: (0, 0)),
        out_shape=jax.ShapeDtypeStruct((2, 256), f32),
    )(oa, oa, ob, ob, bias1_2d)


def _k3b_body(oa0_ref, oa1_ref, ob0_ref, ob1_ref, b1_ref, stats_ref,
              g_ref, bt_ref, w2_ref, as2_ref, ad2_ref, h2_ref, aa2_ref):
    i = pl.program_id(0)
    x1 = jnp.concatenate([oa0_ref[...], oa1_ref[...], ob0_ref[...],
                          ob1_ref[...]], axis=1) + b1_ref[...]
    m = stats_ref[0:1, :] / N
    v = stats_ref[1:2, :] / N - m * m
    xn = (x1 - m) / jnp.sqrt(v + 1e-5) * g_ref[...] + bt_ref[...]
    xe = jnp.where(xn > 0, xn, jnp.exp(xn) - 1.0)
    h2 = jnp.dot(xe, w2_ref[...], preferred_element_type=f32)
    row = i * RB + lax.broadcasted_iota(i32, (RB, 1), 0)
    h2 = jnp.where(row < N, h2, 0.0)
    h2_ref[...] = h2
    as2 = jnp.dot(h2, as2_ref[...].T, preferred_element_type=f32)
    ad2 = jnp.dot(h2, ad2_ref[...].T, preferred_element_type=f32)
    aa2_ref[...] = jnp.concatenate([as2, ad2], axis=1)


def _k3b(oa, ob, bias1_2d, stats, gamma_2d, beta_2d, W2, att_src2, att_dst2):
    return pl.pallas_call(
        _k3b_body,
        grid=(NRB,),
        in_specs=[
            pl.BlockSpec((RB, 64), lambda i: (i, 0)),
            pl.BlockSpec((RB, 64), lambda i: (NRB + i, 0)),
            pl.BlockSpec((RB, 64), lambda i: (i, 0)),
            pl.BlockSpec((RB, 64), lambda i: (NRB + i, 0)),
            pl.BlockSpec((1, 256), lambda i: (0, 0)),
            pl.BlockSpec((2, 256), lambda i: (0, 0)),
            pl.BlockSpec((1, 256), lambda i: (0, 0)),
            pl.BlockSpec((1, 256), lambda i: (0, 0)),
            pl.BlockSpec((256, 2), lambda i: (0, 0)),
            pl.BlockSpec((1, 2), lambda i: (0, 0)),
            pl.BlockSpec((1, 2), lambda i: (0, 0)),
        ],
        out_specs=[
            pl.BlockSpec((RB, 2), lambda i: (i, 0)),
            pl.BlockSpec((RB, 2), lambda i: (i, 0)),
        ],
        out_shape=[
            jax.ShapeDtypeStruct((NPAD, 2), f32),
            jax.ShapeDtypeStruct((NPAD, 2), f32),
        ],
    )(oa, oa, ob, ob, bias1_2d, stats, gamma_2d, beta_2d, W2,
      att_src2, att_dst2)


def _k4_body(srcp, dstp, as2f, ad2f, h2f, z1d, bpad, att2p, out2f,
             as2b, ad2b, h2b, dn2b, sbuf, dbuf,
             exg, ixd, attb, m0, m1, ix0, ix1, obuf, bb, dn2, o2acc):
    cid = lax.axis_index("c")
    sid = lax.axis_index("s")

    @pl.when(cid == 0)
    def _():
        pltpu.sync_copy(as2f, as2b)
        pltpu.sync_copy(ad2f, ad2b)
        pltpu.sync_copy(h2f, h2b)
        pltpu.sync_copy(bpad, bb)
        pltpu.sync_copy(z1d.at[pl.ds(sid * 640, 640)],
                        dn2.at[pl.ds(sid * 640, 640)])
        pltpu.sync_copy(z1d.at[pl.ds(sid * 1280, 1280)],
                        o2acc.at[pl.ds(sid * 1280, 1280)])
        plsc.subcore_barrier()

        base_t = sid * EPT

        def supA(u, _):
            sb0 = base_t + u * SUP
            pltpu.sync_copy(srcp.at[pl.ds(sb0, SUP)], sbuf)
            pltpu.sync_copy(dstp.at[pl.ds(sb0, SUP)], dbuf)

            def grpA(g, _):
                def vA(v, _):
                    off = g * G + v * 16
                    si = sbuf[pl.ds(off, 16)]
                    di = dbuf[pl.ds(off, 16)]
                    a = plsc.load_gather(as2b, [si])
                    b = plsc.load_gather(ad2b, [di])
                    al = a + b
                    al = jnp.maximum(al, 0.2 * al)
                    exg[pl.ds(v * 16, 16)] = jnp.exp(al)
                    ixd[pl.ds(v * 16, 16)] = di
                    return 0

                lax.fori_loop(0, 8, vA, 0)
                pltpu.sync_copy(exg, dn2.at[ixd], add=True)
                return 0

            lax.fori_loop(0, NG, grpA, 0)
            return 0

        lax.fori_loop(0, NSUP, supA, 0)
        plsc.subcore_barrier()
        pltpu.sync_copy(dn2, dn2b)

        def supB(u, _):
            sb0 = base_t + u * SUP
            pltpu.sync_copy(srcp.at[pl.ds(sb0, SUP)], sbuf)
            pltpu.sync_copy(dstp.at[pl.ds(sb0, SUP)], dbuf)

            def grpB(g, _):
                def vB(v, _):
                    off = g * G + v * 16
                    si = sbuf[pl.ds(off, 16)]
                    di = dbuf[pl.ds(off, 16)]
                    a = plsc.load_gather(as2b, [si])
                    b = plsc.load_gather(ad2b, [di])
                    al = a + b
                    al = jnp.maximum(al, 0.2 * al)
                    ex = jnp.exp(al)
                    dn = plsc.load_gather(dn2b, [di])
                    att = ex / (dn + 1e-16)
                    attb[pl.ds(v * 16, 16)] = att
                    h20 = plsc.load_gather(h2b, [2 * si])
                    h21 = plsc.load_gather(h2b, [2 * si + 1])
                    m0[pl.ds(v * 16, 16)] = att * h20
                    m1[pl.ds(v * 16, 16)] = att * h21
                    ix0[pl.ds(v * 16, 16)] = 2 * di
                    ix1[pl.ds(v * 16, 16)] = 2 * di + 1
                    return 0

                lax.fori_loop(0, 8, vB, 0)
                pltpu.sync_copy(attb, att2p.at[pl.ds(base_t + u * SUP + g * G, G)])
                pltpu.sync_copy(m0, o2acc.at[ix0], add=True)
                pltpu.sync_copy(m1, o2acc.at[ix1], add=True)
                return 0

            lax.fori_loop(0, NG, grpB, 0)
            return 0

        lax.fori_loop(0, NSUP, supB, 0)
        plsc.subcore_barrier()

        o0 = sid * 1280
        pltpu.sync_copy(o2acc.at[pl.ds(o0, 1280)], obuf)
        bpv = bb[pl.ds(0, 16)]

        def addb(k, _):
            obuf[pl.ds(k * 16, 16)] = obuf[pl.ds(k * 16, 16)] + bpv
            return 0

        lax.fori_loop(0, 80, addb, 0)
        pltpu.sync_copy(obuf, out2f.at[pl.ds(o0, 1280)])


def _k4(srcp, dstp, as2f, ad2f, h2f, z1d, bpad):
    mesh = plsc.VectorSubcoreMesh(core_axis_name="c", subcore_axis_name="s", num_cores=2, num_subcores=16)
    return pl.kernel(
        _k4_body,
        out_type=[
            jax.ShapeDtypeStruct((EPAD,), f32),
            jax.ShapeDtypeStruct((2 * NPAD,), f32),
        ],
        mesh=mesh,
        compiler_params=pltpu.CompilerParams(needs_layout_passes=False, use_tc_tiling_on_sc=False),
        scratch_types=[
            pltpu.VMEM((NPAD,), f32),
            pltpu.VMEM((NPAD,), f32),
            pltpu.VMEM((2 * NPAD,), f32),
            pltpu.VMEM((NPAD,), f32),
            pltpu.VMEM((SUP,), i32),
            pltpu.VMEM((SUP,), i32),
            pltpu.VMEM((G,), f32),
            pltpu.VMEM((G,), i32),
            pltpu.VMEM((G,), f32),
            pltpu.VMEM((G,), f32),
            pltpu.VMEM((G,), f32),
            pltpu.VMEM((G,), i32),
            pltpu.VMEM((G,), i32),
            pltpu.VMEM((1280,), f32),
            pltpu.VMEM((16,), f32),
            pltpu.VMEM_SHARED((NPAD,), f32),
            pltpu.VMEM_SHARED((2 * NPAD,), f32),
        ],
    )(srcp, dstp, as2f, ad2f, h2f, z1d, bpad)


def kernel(x, edge_index, W1, att_src1, att_dst1, bias1, gamma, beta,
           W2, att_src2, att_dst2, bias2):
    x_pad = jnp.pad(x, ((0, NPAD - N), (0, 0)))
    loop = jnp.arange(N, dtype=jnp.int32)
    padv = jnp.full((EPAD - EP,), N, dtype=jnp.int32)
    srcp = jnp.concatenate([edge_index[0].astype(jnp.int32), loop, padv])
    dstp = jnp.concatenate([edge_index[1].astype(jnp.int32), loop, padv])

    h4, asad = _k1(x_pad, W1, att_src1, att_dst1)
    h4f = h4.reshape(4 * NPAD, 64)
    as_all = asad[:, 0:4].T.reshape(-1)
    ad_all = asad[:, 4:8].T.reshape(-1)

    z2d = jnp.zeros((NPAD, 64), f32)
    z1d = jnp.zeros((2 * NPAD,), f32)
    oa = _k2(0, srcp, dstp, as_all, ad_all, h4f, z2d, z1d)
    z2d_b = z2d + 0.0 * oa[0:NPAD]
    ob = _k2(1, srcp, dstp, as_all, ad_all, h4f, z2d_b, z1d)

    bias1_2d = bias1.reshape(1, 256)
    stats = _k3a(oa, ob, bias1_2d)
    h2m, aa2 = _k3b(oa, ob, bias1_2d, stats, gamma.reshape(1, 256),
                    beta.reshape(1, 256), W2, att_src2, att_dst2)

    as2f = aa2[:, 0] + 0.0
    ad2f = aa2[:, 1] + 0.0
    h2f = h2m.reshape(-1)
    bpad = jnp.tile(bias2, 8)

    att2p, out2f = _k4(srcp, dstp, as2f, ad2f, h2f, z1d, bpad)

    x2 = out2f.reshape(NPAD, 2)[:N]
    att2 = att2p[:EP].reshape(EP, 1)
    return (x2, att2)


# speedup vs baseline: 36.3335x; 1.2320x over previous
import functools

import jax
import jax.numpy as jnp
from jax import lax
from jax.experimental import pallas as pl
from jax.experimental.pallas import tpu as pltpu
from jax.experimental.pallas import tpu_sc as plsc

N = 10000
E = 320000
EP = 330000
IN_DIM = 128
HID = 64
HEADS = 4
OUT = 2

NPAD = 10240
EPAD = 331776
EPT = EPAD // 16
SUP = 6912
NSUP = EPT // SUP
G = 128
NG = SUP // G
RB = 1280
NRB = NPAD // RB

f32 = jnp.float32
i32 = jnp.int32


def _k1_body(x_ref, w1_ref, asrc_ref, adst_ref, h4_ref, asad_ref):
    h = jnp.dot(x_ref[...], w1_ref[...], preferred_element_type=f32)
    for hh in range(4):
        h4_ref[hh] = h[:, hh * 64:(hh + 1) * 64]
    hr = h.reshape(RB, HEADS, HID)
    a_s = (hr * asrc_ref[...][None]).sum(-1)
    a_d = (hr * adst_ref[...][None]).sum(-1)
    asad_ref[...] = jnp.concatenate([a_s, a_d], axis=1)


def _k1(x_pad, W1, att_src1, att_dst1):
    return pl.pallas_call(
        _k1_body,
        grid=(NRB,),
        in_specs=[
            pl.BlockSpec((RB, IN_DIM), lambda i: (i, 0)),
            pl.BlockSpec((IN_DIM, HEADS * HID), lambda i: (0, 0)),
            pl.BlockSpec((HEADS, HID), lambda i: (0, 0)),
            pl.BlockSpec((HEADS, HID), lambda i: (0, 0)),
        ],
        out_specs=[
            pl.BlockSpec((4, RB, 64), lambda i: (0, i, 0)),
            pl.BlockSpec((RB, 8), lambda i: (i, 0)),
        ],
        out_shape=[
            jax.ShapeDtypeStruct((4, NPAD, 64), f32),
            jax.ShapeDtypeStruct((NPAD, 8), f32),
        ],
    )(x_pad, W1, att_src1, att_dst1)


def _k2_body(j, srcp, dstp, asall, adall, h4, z2d, z1d, outj,
             asb, adb, dnb, sbuf, dbuf,
             sg0, sg1, dg0, dg1, ex0, ex1, ix0, ix1, at0, at1, hr0, hr1,
             sA0, sA1, sG0, sG1, sS0, sS1, acc, dnm):
    cid = lax.axis_index("c")
    sid = lax.axis_index("s")
    hsel = 2 * j + cid
    rows_pt = NPAD // 16
    r0 = sid * rows_pt

    pltpu.sync_copy(asall.at[pl.ds(hsel * NPAD, NPAD)], asb)
    pltpu.sync_copy(adall.at[pl.ds(hsel * NPAD, NPAD)], adb)
    base_t = sid * EPT
    pltpu.sync_copy(z2d.at[pl.ds(r0, rows_pt)], acc.at[pl.ds(r0, rows_pt)])
    pltpu.sync_copy(z1d.at[pl.ds(sid * 640, 640)],
                    dnm.at[pl.ds(sid * 640, 640)])
    plsc.subcore_barrier()

    def grpA(g, exg, ixg, sem):
        def vA(v, _):
            off = g * G + v * 16
            si = sbuf[pl.ds(off, 16)]
            di = dbuf[pl.ds(off, 16)]
            a = plsc.load_gather(asb, [si])
            b = plsc.load_gather(adb, [di])
            al = a + b
            al = jnp.maximum(al, 0.2 * al)
            exg[pl.ds(v * 16, 16)] = jnp.exp(al)
            ixg[pl.ds(v * 16, 16)] = di
            return 0

        lax.fori_loop(0, 8, vA, 0)
        return pltpu.async_copy(exg, dnm.at[ixg], sem, add=True)

    def pairA(gg, _):
        cpy0 = grpA(2 * gg, ex0, ix0, sA0)
        cpy1 = grpA(2 * gg + 1, ex1, ix1, sA1)
        cpy0.wait()
        cpy1.wait()
        return 0

    def supA(u, _):
        sb0 = base_t + u * SUP
        pltpu.sync_copy(srcp.at[pl.ds(sb0, SUP)], sbuf)
        pltpu.sync_copy(dstp.at[pl.ds(sb0, SUP)], dbuf)
        lax.fori_loop(0, NG // 2, pairA, 0)
        return 0

    lax.fori_loop(0, NSUP, supA, 0)
    plsc.subcore_barrier()
    pltpu.sync_copy(dnm, dnb)

    def attB(g, sg, dg, attg):
        def vB(v, _):
            off = g * G + v * 16
            si = sbuf[pl.ds(off, 16)]
            di = dbuf[pl.ds(off, 16)]
            sg[pl.ds(v * 16, 16)] = si + hsel * NPAD
            dg[pl.ds(v * 16, 16)] = di
            a = plsc.load_gather(asb, [si])
            b = plsc.load_gather(adb, [di])
            al = a + b
            al = jnp.maximum(al, 0.2 * al)
            ex = jnp.exp(al)
            dn = plsc.load_gather(dnb, [di])
            attg[pl.ds(v * 16, 16)] = ex / (dn + 1e-16)
            return 0

        lax.fori_loop(0, 8, vB, 0)

    def scaleB(attg, hrows):
        def eB(e, _):
            zi = jnp.zeros((16,), i32)
            a0 = plsc.load_gather(attg, [zi + e])
            for q in range(4):
                hv = hrows[e, pl.ds(q * 16, 16)]
                hrows[e, pl.ds(q * 16, 16)] = hv * a0
            return 0

        lax.fori_loop(0, G, eB, 0)

    def pairB(gg, _):
        g0 = 2 * gg
        g1 = 2 * gg + 1
        attB(g0, sg0, dg0, at0)
        gat0 = pltpu.async_copy(h4.at[sg0], hr0, sG0)
        attB(g1, sg1, dg1, at1)
        gat1 = pltpu.async_copy(h4.at[sg1], hr1, sG1)
        gat0.wait()
        scaleB(at0, hr0)
        sc0 = pltpu.async_copy(hr0, acc.at[dg0], sS0, add=True)
        gat1.wait()
        scaleB(at1, hr1)
        sc1 = pltpu.async_copy(hr1, acc.at[dg1], sS1, add=True)
        sc0.wait()
        sc1.wait()
        return 0

    def supB(u, _):
        sb0 = base_t + u * SUP
        pltpu.sync_copy(srcp.at[pl.ds(sb0, SUP)], sbuf)
        pltpu.sync_copy(dstp.at[pl.ds(sb0, SUP)], dbuf)
        lax.fori_loop(0, NG // 2, pairB, 0)
        return 0

    lax.fori_loop(0, NSUP, supB, 0)
    plsc.subcore_barrier()
    pltpu.sync_copy(acc.at[pl.ds(r0, rows_pt)],
                    outj.at[pl.ds(cid * NPAD + r0, rows_pt)])


def _k2(j, srcp, dstp, as_all, ad_all, h4, z2d, z1d):
    mesh = plsc.VectorSubcoreMesh(core_axis_name="c", subcore_axis_name="s", num_cores=2, num_subcores=16)
    return pl.kernel(
        functools.partial(_k2_body, j),
        out_type=jax.ShapeDtypeStruct((2 * NPAD, 64), f32),
        mesh=mesh,
        compiler_params=pltpu.CompilerParams(needs_layout_passes=False, use_tc_tiling_on_sc=False),
        scratch_types=[
            pltpu.VMEM((NPAD,), f32),
            pltpu.VMEM((NPAD,), f32),
            pltpu.VMEM((NPAD,), f32),
            pltpu.VMEM((SUP,), i32),
            pltpu.VMEM((SUP,), i32),
            pltpu.VMEM((G,), i32),
            pltpu.VMEM((G,), i32),
            pltpu.VMEM((G,), i32),
            pltpu.VMEM((G,), i32),
            pltpu.VMEM((G,), f32),
            pltpu.VMEM((G,), f32),
            pltpu.VMEM((G,), i32),
            pltpu.VMEM((G,), i32),
            pltpu.VMEM((G,), f32),
            pltpu.VMEM((G,), f32),
            pltpu.VMEM((G, 64), f32),
            pltpu.VMEM((G, 64), f32),
            pltpu.SemaphoreType.DMA,
            pltpu.SemaphoreType.DMA,
            pltpu.SemaphoreType.DMA,
            pltpu.SemaphoreType.DMA,
            pltpu.SemaphoreType.DMA,
            pltpu.SemaphoreType.DMA,
            pltpu.VMEM_SHARED((NPAD, 64), f32),
            pltpu.VMEM_SHARED((NPAD,), f32),
        ],
    )(srcp, dstp, as_all, ad_all, h4, z2d, z1d)


def _k3a_body(oa0_ref, oa1_ref, ob0_ref, ob1_ref, b1_ref, stats_ref):
    i = pl.program_id(0)
    x1 = jnp.concatenate([oa0_ref[...], oa1_ref[...], ob0_ref[...],
                          ob1_ref[...]], axis=1) + b1_ref[...]
    row = i * RB + lax.broadcasted_iota(i32, (RB, 1), 0)
    x1 = jnp.where(row < N, x1, 0.0)
    s1 = jnp.sum(x1, axis=0, keepdims=True)
    s2 = jnp.sum(x1 * x1, axis=0, keepdims=True)
    blk = jnp.concatenate([s1, s2], axis=0)

    @pl.when(i == 0)
    def _():
        stats_ref[...] = jnp.zeros_like(stats_ref)

    stats_ref[...] += blk


def _k3a(oa, ob, bias1_2d):
    return pl.pallas_call(
        _k3a_body,
        grid=(NRB,),
        in_specs=[
            pl.BlockSpec((RB, 64), lambda i: (i, 0)),
            pl.BlockSpec((RB, 64), lambda i: (NRB + i, 0)),
            pl.BlockSpec((RB, 64), lambda i: (i, 0)),
            pl.BlockSpec((RB, 64), lambda i: (NRB + i, 0)),
            pl.BlockSpec((1, 256), lambda i: (0, 0)),
        ],
        out_specs=pl.BlockSpec((2, 256), lambda i: (0, 0)),
        out_shape=jax.ShapeDtypeStruct((2, 256), f32),
    )(oa, oa, ob, ob, bias1_2d)


def _k3b_body(oa0_ref, oa1_ref, ob0_ref, ob1_ref, b1_ref, stats_ref,
              g_ref, bt_ref, w2_ref, as2_ref, ad2_ref, h2_ref, aa2_ref):
    i = pl.program_id(0)
    x1 = jnp.concatenate([oa0_ref[...], oa1_ref[...], ob0_ref[...],
                          ob1_ref[...]], axis=1) + b1_ref[...]
    m = stats_ref[0:1, :] / N
    v = stats_ref[1:2, :] / N - m * m
    xn = (x1 - m) / jnp.sqrt(v + 1e-5) * g_ref[...] + bt_ref[...]
    xe = jnp.where(xn > 0, xn, jnp.exp(xn) - 1.0)
    h2 = jnp.dot(xe, w2_ref[...], preferred_element_type=f32)
    row = i * RB + lax.broadcasted_iota(i32, (RB, 1), 0)
    h2 = jnp.where(row < N, h2, 0.0)
    h2_ref[...] = h2
    as2 = jnp.dot(h2, as2_ref[...].T, preferred_element_type=f32)
    ad2 = jnp.dot(h2, ad2_ref[...].T, preferred_element_type=f32)
    aa2_ref[...] = jnp.concatenate([as2, ad2], axis=1)


def _k3b(oa, ob, bias1_2d, stats, gamma_2d, beta_2d, W2, att_src2, att_dst2):
    return pl.pallas_call(
        _k3b_body,
        grid=(NRB,),
        in_specs=[
            pl.BlockSpec((RB, 64), lambda i: (i, 0)),
            pl.BlockSpec((RB, 64), lambda i: (NRB + i, 0)),
            pl.BlockSpec((RB, 64), lambda i: (i, 0)),
            pl.BlockSpec((RB, 64), lambda i: (NRB + i, 0)),
            pl.BlockSpec((1, 256), lambda i: (0, 0)),
            pl.BlockSpec((2, 256), lambda i: (0, 0)),
            pl.BlockSpec((1, 256), lambda i: (0, 0)),
            pl.BlockSpec((1, 256), lambda i: (0, 0)),
            pl.BlockSpec((256, 2), lambda i: (0, 0)),
            pl.BlockSpec((1, 2), lambda i: (0, 0)),
            pl.BlockSpec((1, 2), lambda i: (0, 0)),
        ],
        out_specs=[
            pl.BlockSpec((RB, 2), lambda i: (i, 0)),
            pl.BlockSpec((RB, 2), lambda i: (i, 0)),
        ],
        out_shape=[
            jax.ShapeDtypeStruct((NPAD, 2), f32),
            jax.ShapeDtypeStruct((NPAD, 2), f32),
        ],
    )(oa, oa, ob, ob, bias1_2d, stats, gamma_2d, beta_2d, W2,
      att_src2, att_dst2)


def _k4_body(srcp, dstp, as2f, ad2f, h2f, z1d, bpad, att2p, out2f,
             as2b, ad2b, h2b, dn2b, sbuf, dbuf,
             exg, ixd, attb, m0, m1, ix0, ix1, obuf, bb, dn2, o2acc):
    cid = lax.axis_index("c")
    sid = lax.axis_index("s")

    @pl.when(cid == 0)
    def _():
        pltpu.sync_copy(as2f, as2b)
        pltpu.sync_copy(ad2f, ad2b)
        pltpu.sync_copy(h2f, h2b)
        pltpu.sync_copy(bpad, bb)
        pltpu.sync_copy(z1d.at[pl.ds(sid * 640, 640)],
                        dn2.at[pl.ds(sid * 640, 640)])
        pltpu.sync_copy(z1d.at[pl.ds(sid * 1280, 1280)],
                        o2acc.at[pl.ds(sid * 1280, 1280)])
        plsc.subcore_barrier()

        base_t = sid * EPT

        def supA(u, _):
            sb0 = base_t + u * SUP
            pltpu.sync_copy(srcp.at[pl.ds(sb0, SUP)], sbuf)
            pltpu.sync_copy(dstp.at[pl.ds(sb0, SUP)], dbuf)

            def grpA(g, _):
                def vA(v, _):
                    off = g * G + v * 16
                    si = sbuf[pl.ds(off, 16)]
                    di = dbuf[pl.ds(off, 16)]
                    a = plsc.load_gather(as2b, [si])
                    b = plsc.load_gather(ad2b, [di])
                    al = a + b
                    al = jnp.maximum(al, 0.2 * al)
                    exg[pl.ds(v * 16, 16)] = jnp.exp(al)
                    ixd[pl.ds(v * 16, 16)] = di
                    return 0

                lax.fori_loop(0, 8, vA, 0)
                pltpu.sync_copy(exg, dn2.at[ixd], add=True)
                return 0

            lax.fori_loop(0, NG, grpA, 0)
            return 0

        lax.fori_loop(0, NSUP, supA, 0)
        plsc.subcore_barrier()
        pltpu.sync_copy(dn2, dn2b)

        def supB(u, _):
            sb0 = base_t + u * SUP
            pltpu.sync_copy(srcp.at[pl.ds(sb0, SUP)], sbuf)
            pltpu.sync_copy(dstp.at[pl.ds(sb0, SUP)], dbuf)

            def grpB(g, _):
                def vB(v, _):
                    off = g * G + v * 16
                    si = sbuf[pl.ds(off, 16)]
                    di = dbuf[pl.ds(off, 16)]
                    a = plsc.load_gather(as2b, [si])
                    b = plsc.load_gather(ad2b, [di])
                    al = a + b
                    al = jnp.maximum(al, 0.2 * al)
                    ex = jnp.exp(al)
                    dn = plsc.load_gather(dn2b, [di])
                    att = ex / (dn + 1e-16)
                    attb[pl.ds(v * 16, 16)] = att
                    h20 = plsc.load_gather(h2b, [2 * si])
                    h21 = plsc.load_gather(h2b, [2 * si + 1])
                    m0[pl.ds(v * 16, 16)] = att * h20
                    m1[pl.ds(v * 16, 16)] = att * h21
                    ix0[pl.ds(v * 16, 16)] = 2 * di
                    ix1[pl.ds(v * 16, 16)] = 2 * di + 1
                    return 0

                lax.fori_loop(0, 8, vB, 0)
                pltpu.sync_copy(attb, att2p.at[pl.ds(base_t + u * SUP + g * G, G)])
                pltpu.sync_copy(m0, o2acc.at[ix0], add=True)
                pltpu.sync_copy(m1, o2acc.at[ix1], add=True)
                return 0

            lax.fori_loop(0, NG, grpB, 0)
            return 0

        lax.fori_loop(0, NSUP, supB, 0)
        plsc.subcore_barrier()

        o0 = sid * 1280
        pltpu.sync_copy(o2acc.at[pl.ds(o0, 1280)], obuf)
        bpv = bb[pl.ds(0, 16)]

        def addb(k, _):
            obuf[pl.ds(k * 16, 16)] = obuf[pl.ds(k * 16, 16)] + bpv
            return 0

        lax.fori_loop(0, 80, addb, 0)
        pltpu.sync_copy(obuf, out2f.at[pl.ds(o0, 1280)])


def _k4(srcp, dstp, as2f, ad2f, h2f, z1d, bpad):
    mesh = plsc.VectorSubcoreMesh(core_axis_name="c", subcore_axis_name="s", num_cores=2, num_subcores=16)
    return pl.kernel(
        _k4_body,
        out_type=[
            jax.ShapeDtypeStruct((EPAD,), f32),
            jax.ShapeDtypeStruct((2 * NPAD,), f32),
        ],
        mesh=mesh,
        compiler_params=pltpu.CompilerParams(needs_layout_passes=False, use_tc_tiling_on_sc=False),
        scratch_types=[
            pltpu.VMEM((NPAD,), f32),
            pltpu.VMEM((NPAD,), f32),
            pltpu.VMEM((2 * NPAD,), f32),
            pltpu.VMEM((NPAD,), f32),
            pltpu.VMEM((SUP,), i32),
            pltpu.VMEM((SUP,), i32),
            pltpu.VMEM((G,), f32),
            pltpu.VMEM((G,), i32),
            pltpu.VMEM((G,), f32),
            pltpu.VMEM((G,), f32),
            pltpu.VMEM((G,), f32),
            pltpu.VMEM((G,), i32),
            pltpu.VMEM((G,), i32),
            pltpu.VMEM((1280,), f32),
            pltpu.VMEM((16,), f32),
            pltpu.VMEM_SHARED((NPAD,), f32),
            pltpu.VMEM_SHARED((2 * NPAD,), f32),
        ],
    )(srcp, dstp, as2f, ad2f, h2f, z1d, bpad)


def kernel(x, edge_index, W1, att_src1, att_dst1, bias1, gamma, beta,
           W2, att_src2, att_dst2, bias2):
    x_pad = jnp.pad(x, ((0, NPAD - N), (0, 0)))
    loop = jnp.arange(N, dtype=jnp.int32)
    padv = jnp.full((EPAD - EP,), N, dtype=jnp.int32)
    srcp = jnp.concatenate([edge_index[0].astype(jnp.int32), loop, padv])
    dstp = jnp.concatenate([edge_index[1].astype(jnp.int32), loop, padv])

    h4, asad = _k1(x_pad, W1, att_src1, att_dst1)
    h4f = h4.reshape(4 * NPAD, 64)
    as_all = asad[:, 0:4].T.reshape(-1)
    ad_all = asad[:, 4:8].T.reshape(-1)

    z2d = jnp.zeros((NPAD, 64), f32)
    z1d = jnp.zeros((2 * NPAD,), f32)
    oa = _k2(0, srcp, dstp, as_all, ad_all, h4f, z2d, z1d)
    z2d_b = z2d + 0.0 * oa[0:NPAD]
    ob = _k2(1, srcp, dstp, as_all, ad_all, h4f, z2d_b, z1d)

    bias1_2d = bias1.reshape(1, 256)
    stats = _k3a(oa, ob, bias1_2d)
    h2m, aa2 = _k3b(oa, ob, bias1_2d, stats, gamma.reshape(1, 256),
                    beta.reshape(1, 256), W2, att_src2, att_dst2)

    as2f = aa2[:, 0] + 0.0
    ad2f = aa2[:, 1] + 0.0
    h2f = h2m.reshape(-1)
    bpad = jnp.tile(bias2, 8)

    att2p, out2f = _k4(srcp, dstp, as2f, ad2f, h2f, z1d, bpad)

    x2 = out2f.reshape(NPAD, 2)[:N]
    att2 = att2p[:EP].reshape(EP, 1)
    return (x2, att2)
